# baseline (speedup 1.0000x reference)
import functools

import jax
import jax.numpy as jnp
from jax import lax
from jax.experimental import pallas as pl
from jax.experimental.pallas import tpu as pltpu
from jax.experimental.pallas import tpu_sc as plsc

DIM = 1024
HEADS = 4
NUM_KEYS = 256
TOPK = 16
DIM_HEAD = 512
DIM_QUERY = DIM_HEAD * HEADS
T = 2048
PK = DIM_HEAD // 2
NJ = 2 * HEADS

TBLK = 256
NEG = -1e30


def _topk16(scores, payload=None):
  rows = scores.shape[0]
  iota = lax.broadcasted_iota(jnp.int32, (rows, NUM_KEYS), 1)
  cur = scores
  vs, is_, ps = [], [], []
  for _ in range(TOPK):
    m = jnp.max(cur, axis=1, keepdims=True)
    sel = cur == m
    pos = jnp.min(jnp.where(sel, iota, NUM_KEYS), axis=1, keepdims=True)
    hit = iota == pos
    vs.append(m)
    is_.append(pos)
    if payload is not None:
      ps.append(jnp.max(jnp.where(hit, payload, -1.0), axis=1, keepdims=True))
    cur = jnp.where(hit, NEG, cur)
  vals = jnp.concatenate(vs, axis=1)
  idx = jnp.concatenate(is_, axis=1)
  if payload is not None:
    return vals, idx, jnp.concatenate(ps, axis=1)
  return vals, idx


def _score_kernel(x_ref, wq_ref, g_ref, b_ref, kt_ref, vidx_ref, wexp_ref):
  q = jnp.dot(x_ref[...].astype(jnp.bfloat16),
              wq_ref[...].astype(jnp.bfloat16),
              preferred_element_type=jnp.float32)
  mu = jnp.mean(q, axis=1, keepdims=True)
  d = q - mu
  var = jnp.mean(d * d, axis=1, keepdims=True)
  q = d * lax.rsqrt(var + 1e-5) * g_ref[...] + b_ref[...]

  q16 = q.astype(jnp.bfloat16)
  s_list, i_list = [], []
  for j in range(NJ):
    dots = jnp.dot(q16[:, j * PK:(j + 1) * PK], kt_ref[j].astype(jnp.bfloat16),
                   preferred_element_type=jnp.float32)
    s, i_ = _topk16(dots)
    s_list.append(s)
    i_list.append(i_)

  li = lax.broadcasted_iota(jnp.int32, (TOPK, NUM_KEYS), 1)
  ri = lax.broadcasted_iota(jnp.int32, (TOPK, NUM_KEYS), 0)
  R = (li // TOPK == ri).astype(jnp.float32)
  C = (li % TOPK == ri).astype(jnp.float32)
  iota16 = lax.broadcasted_iota(jnp.int32, (TBLK, TOPK), 1)

  for h in range(HEADS):
    s0, s1 = s_list[h], s_list[HEADS + h]
    i0, i1 = i_list[h], i_list[HEADS + h]
    hi = lax.Precision.HIGHEST
    all_s = (jnp.dot(s0, R, preferred_element_type=jnp.float32, precision=hi)
             + jnp.dot(s1, C, preferred_element_type=jnp.float32, precision=hi))
    fs, cpos = _topk16(all_s)
    vis = []
    for k in range(TOPK):
      ck = cpos[:, k:k + 1]
      i0s = jnp.sum(jnp.where(iota16 == ck // TOPK, i0, 0),
                    axis=1, keepdims=True)
      i1s = jnp.sum(jnp.where(iota16 == ck % TOPK, i1, 0),
                    axis=1, keepdims=True)
      vis.append(i0s * NUM_KEYS + i1s)
    e = jnp.exp(fs - fs[:, :1])
    w = e / jnp.sum(e, axis=1, keepdims=True)
    vidx_ref[:, h * TOPK:(h + 1) * TOPK] = jnp.concatenate(vis, axis=1)
    wexp_ref[:, h * PK:(h + 1) * PK] = jnp.dot(
        w, R, preferred_element_type=jnp.float32, precision=lax.Precision.HIGHEST)


def _tc_score(x2d, Wq, g2d, b2d, Kt):
  nblk = T // TBLK
  return pl.pallas_call(
      _score_kernel,
      grid=(nblk,),
      in_specs=[
          pl.BlockSpec((TBLK, DIM), lambda i: (i, 0)),
          pl.BlockSpec((DIM, DIM_QUERY), lambda i: (0, 0)),
          pl.BlockSpec((1, DIM_QUERY), lambda i: (0, 0)),
          pl.BlockSpec((1, DIM_QUERY), lambda i: (0, 0)),
          pl.BlockSpec((NJ, PK, NUM_KEYS), lambda i: (0, 0, 0)),
      ],
      out_specs=[
          pl.BlockSpec((TBLK, HEADS * TOPK), lambda i: (i, 0)),
          pl.BlockSpec((TBLK, HEADS * TOPK * 16), lambda i: (i, 0)),
      ],
      out_shape=[
          jax.ShapeDtypeStruct((T, HEADS * TOPK), jnp.int32),
          jax.ShapeDtypeStruct((T, HEADS * TOPK * 16), jnp.float32),
      ],
  )(x2d, Wq, g2d, b2d, Kt)


K_PER_T = HEADS * TOPK


HALF_ROWS = K_PER_T // 2


def _bag_body(vidx_hbm, attn_hbm, values_hbm, out_hbm,
              idx2_v, w2_v, bufa_v, bufb_v, acc_v, sema, semb):
  nc = 2
  wid = lax.axis_index("s") * nc + lax.axis_index("c")
  t_per_w = T // 32
  t0 = wid * t_per_w

  def ga(parity):
    return pltpu.make_async_copy(
        values_hbm.at[idx2_v.at[parity, pl.ds(0, HALF_ROWS)]], bufa_v, sema)

  def gb(parity):
    return pltpu.make_async_copy(
        values_hbm.at[idx2_v.at[parity, pl.ds(HALF_ROWS, HALF_ROWS)]],
        bufb_v, semb)

  def accumulate(buf, p, lane0, init):
    out = []
    for c in range(4):
      def rstep(r, acc):
        wb = w2_v[p, pl.ds(lane0 + r * 16, 16)]
        return tuple(
            acc[dv] + wb * buf[r, pl.ds(c * 256 + dv * 16, 16)]
            for dv in range(16))
      if init is None:
        acc0 = tuple(jnp.zeros((16,), jnp.float32) for _ in range(16))
      else:
        acc0 = tuple(acc_v[pl.ds(c * 256 + dv * 16, 16)] for dv in range(16))
      acc = lax.fori_loop(0, HALF_ROWS, rstep, acc0)
      for dv in range(16):
        acc_v[pl.ds(c * 256 + dv * 16, 16)] = acc[dv]
    del out

  pltpu.sync_copy(vidx_hbm.at[t0], idx2_v.at[0])
  pltpu.sync_copy(attn_hbm.at[t0], w2_v.at[0])
  ga(0).start()

  def token(i, carry):
    t = t0 + i
    p = lax.rem(i, 2)
    pn = lax.rem(i + 1, 2)
    tn = t0 + lax.rem(i + 1, t_per_w)
    pltpu.sync_copy(vidx_hbm.at[tn], idx2_v.at[pn])
    pltpu.sync_copy(attn_hbm.at[tn], w2_v.at[pn])
    gb(p).start()
    ga(p).wait()
    accumulate(bufa_v, p, 0, init=None)
    ga(pn).start()
    gb(p).wait()
    accumulate(bufb_v, p, HALF_ROWS * 16, init=acc_v)
    pltpu.sync_copy(acc_v, out_hbm.at[t])
    return carry

  lax.fori_loop(0, t_per_w, token, 0)
  ga(0).wait()


def _sc_bag(vidx, attn, values):
  mesh = plsc.VectorSubcoreMesh(core_axis_name="c", subcore_axis_name="s")
  f = pl.kernel(
      _bag_body,
      out_type=jax.ShapeDtypeStruct((T, DIM), jnp.float32),
      mesh=mesh,
      scratch_types=[
          pltpu.VMEM((2, K_PER_T), jnp.int32),
          pltpu.VMEM((2, K_PER_T * 16), jnp.float32),
          pltpu.VMEM((HALF_ROWS, DIM), jnp.float32),
          pltpu.VMEM((HALF_ROWS, DIM), jnp.float32),
          pltpu.VMEM((DIM,), jnp.float32),
          pltpu.SemaphoreType.DMA,
          pltpu.SemaphoreType.DMA,
      ],
  )
  return f(vidx, attn, values)


def kernel(x, Wq, ln_g, ln_b, keys, values):
  t, b, e = x.shape
  x2d = x.reshape(t * b, e)
  g2d = ln_g.reshape(1, DIM_QUERY)
  b2d = ln_b.reshape(1, DIM_QUERY)
  Kt = jnp.transpose(keys, (2, 0, 3, 1)).reshape(NJ, PK, NUM_KEYS)
  vidx, attn = _tc_score(x2d, Wq, g2d, b2d, Kt)
  out = _sc_bag(vidx, attn, values)
  return out.reshape(t, b, e)


# speedup vs baseline: 1.1665x; 1.1665x over previous
import functools

import jax
import jax.numpy as jnp
from jax import lax
from jax.experimental import pallas as pl
from jax.experimental.pallas import tpu as pltpu
from jax.experimental.pallas import tpu_sc as plsc

DIM = 1024
HEADS = 4
NUM_KEYS = 256
TOPK = 16
DIM_HEAD = 512
DIM_QUERY = DIM_HEAD * HEADS
T = 2048
PK = DIM_HEAD // 2
NJ = 2 * HEADS

TBLK = 256
NEG = -1e30


def _topk16(scores, payload=None):
  rows = scores.shape[0]
  iota = lax.broadcasted_iota(jnp.int32, (rows, NUM_KEYS), 1)
  cur = scores
  vs, is_, ps = [], [], []
  for _ in range(TOPK):
    m = jnp.max(cur, axis=1, keepdims=True)
    sel = cur == m
    pos = jnp.min(jnp.where(sel, iota, NUM_KEYS), axis=1, keepdims=True)
    hit = iota == pos
    vs.append(m)
    is_.append(pos)
    if payload is not None:
      ps.append(jnp.max(jnp.where(hit, payload, -1.0), axis=1, keepdims=True))
    cur = jnp.where(hit, NEG, cur)
  vals = jnp.concatenate(vs, axis=1)
  idx = jnp.concatenate(is_, axis=1)
  if payload is not None:
    return vals, idx, jnp.concatenate(ps, axis=1)
  return vals, idx


def _score_kernel(x_ref, wq_ref, g_ref, b_ref, kt_ref, vidx_ref, wexp_ref):
  q = jnp.dot(x_ref[...].astype(jnp.bfloat16),
              wq_ref[...].astype(jnp.bfloat16),
              preferred_element_type=jnp.float32)
  mu = jnp.mean(q, axis=1, keepdims=True)
  d = q - mu
  var = jnp.mean(d * d, axis=1, keepdims=True)
  q = d * lax.rsqrt(var + 1e-5) * g_ref[...] + b_ref[...]

  q16 = q.astype(jnp.bfloat16)
  s_list, i_list = [], []
  for j in range(NJ):
    dots = jnp.dot(q16[:, j * PK:(j + 1) * PK], kt_ref[j].astype(jnp.bfloat16),
                   preferred_element_type=jnp.float32)
    s, i_ = _topk16(dots)
    s_list.append(s)
    i_list.append(i_.astype(jnp.float32))

  li = lax.broadcasted_iota(jnp.int32, (TOPK, NUM_KEYS), 1)
  ri = lax.broadcasted_iota(jnp.int32, (TOPK, NUM_KEYS), 0)
  R = (li // TOPK == ri).astype(jnp.float32)
  C = (li % TOPK == ri).astype(jnp.float32)

  for h in range(HEADS):
    s0, s1 = s_list[h], s_list[HEADS + h]
    i0, i1 = i_list[h], i_list[HEADS + h]
    hi = lax.Precision.HIGHEST
    all_s = (jnp.dot(s0, R, preferred_element_type=jnp.float32, precision=hi)
             + jnp.dot(s1, C, preferred_element_type=jnp.float32, precision=hi))
    all_i = (jnp.dot(i0, R * float(NUM_KEYS), preferred_element_type=jnp.float32,
                     precision=hi)
             + jnp.dot(i1, C, preferred_element_type=jnp.float32, precision=hi))
    fs, _, fi = _topk16(all_s, payload=all_i)
    e = jnp.exp(fs - fs[:, :1])
    w = e / jnp.sum(e, axis=1, keepdims=True)
    vidx_ref[:, h * TOPK:(h + 1) * TOPK] = fi.astype(jnp.int32)
    wexp_ref[:, h * PK:(h + 1) * PK] = jnp.dot(
        w, R, preferred_element_type=jnp.float32, precision=lax.Precision.HIGHEST)


def _tc_score(x2d, Wq, g2d, b2d, Kt):
  nblk = T // TBLK
  return pl.pallas_call(
      _score_kernel,
      grid=(nblk,),
      in_specs=[
          pl.BlockSpec((TBLK, DIM), lambda i: (i, 0)),
          pl.BlockSpec((DIM, DIM_QUERY), lambda i: (0, 0)),
          pl.BlockSpec((1, DIM_QUERY), lambda i: (0, 0)),
          pl.BlockSpec((1, DIM_QUERY), lambda i: (0, 0)),
          pl.BlockSpec((NJ, PK, NUM_KEYS), lambda i: (0, 0, 0)),
      ],
      out_specs=[
          pl.BlockSpec((TBLK, HEADS * TOPK), lambda i: (i, 0)),
          pl.BlockSpec((TBLK, HEADS * TOPK * 16), lambda i: (i, 0)),
      ],
      out_shape=[
          jax.ShapeDtypeStruct((T, HEADS * TOPK), jnp.int32),
          jax.ShapeDtypeStruct((T, HEADS * TOPK * 16), jnp.float32),
      ],
  )(x2d, Wq, g2d, b2d, Kt)


K_PER_T = HEADS * TOPK


HALF_ROWS = K_PER_T // 2


def _bag_body(vidx_hbm, attn_hbm, values_hbm, out_hbm,
              idx2_v, w2_v, bufa_v, bufb_v, acc_v, sema, semb):
  nc = 2
  wid = lax.axis_index("s") * nc + lax.axis_index("c")
  t_per_w = T // 32
  t0 = wid * t_per_w

  def ga(parity):
    return pltpu.make_async_copy(
        values_hbm.at[idx2_v.at[parity, pl.ds(0, HALF_ROWS)]], bufa_v, sema)

  def gb(parity):
    return pltpu.make_async_copy(
        values_hbm.at[idx2_v.at[parity, pl.ds(HALF_ROWS, HALF_ROWS)]],
        bufb_v, semb)

  def accumulate(buf, p, lane0, init):
    out = []
    for c in range(4):
      def rstep(r, acc):
        wb = w2_v[p, pl.ds(lane0 + r * 16, 16)]
        return tuple(
            acc[dv] + wb * buf[r, pl.ds(c * 256 + dv * 16, 16)]
            for dv in range(16))
      if init is None:
        acc0 = tuple(jnp.zeros((16,), jnp.float32) for _ in range(16))
      else:
        acc0 = tuple(acc_v[pl.ds(c * 256 + dv * 16, 16)] for dv in range(16))
      acc = lax.fori_loop(0, HALF_ROWS, rstep, acc0)
      for dv in range(16):
        acc_v[pl.ds(c * 256 + dv * 16, 16)] = acc[dv]
    del out

  pltpu.sync_copy(vidx_hbm.at[t0], idx2_v.at[0])
  pltpu.sync_copy(attn_hbm.at[t0], w2_v.at[0])
  ga(0).start()

  def token(i, carry):
    t = t0 + i
    p = lax.rem(i, 2)
    pn = lax.rem(i + 1, 2)
    tn = t0 + lax.rem(i + 1, t_per_w)
    pltpu.sync_copy(vidx_hbm.at[tn], idx2_v.at[pn])
    pltpu.sync_copy(attn_hbm.at[tn], w2_v.at[pn])
    gb(p).start()
    ga(p).wait()
    accumulate(bufa_v, p, 0, init=None)
    ga(pn).start()
    gb(p).wait()
    accumulate(bufb_v, p, HALF_ROWS * 16, init=acc_v)
    pltpu.sync_copy(acc_v, out_hbm.at[t])
    return carry

  lax.fori_loop(0, t_per_w, token, 0)
  ga(0).wait()


def _sc_bag(vidx, attn, values):
  mesh = plsc.VectorSubcoreMesh(core_axis_name="c", subcore_axis_name="s")
  f = pl.kernel(
      _bag_body,
      out_type=jax.ShapeDtypeStruct((T, DIM), jnp.float32),
      mesh=mesh,
      scratch_types=[
          pltpu.VMEM((2, K_PER_T), jnp.int32),
          pltpu.VMEM((2, K_PER_T * 16), jnp.float32),
          pltpu.VMEM((HALF_ROWS, DIM), jnp.float32),
          pltpu.VMEM((HALF_ROWS, DIM), jnp.float32),
          pltpu.VMEM((DIM,), jnp.float32),
          pltpu.SemaphoreType.DMA,
          pltpu.SemaphoreType.DMA,
      ],
  )
  return f(vidx, attn, values)


def kernel(x, Wq, ln_g, ln_b, keys, values):
  t, b, e = x.shape
  x2d = x.reshape(t * b, e)
  g2d = ln_g.reshape(1, DIM_QUERY)
  b2d = ln_b.reshape(1, DIM_QUERY)
  Kt = jnp.transpose(keys, (2, 0, 3, 1)).reshape(NJ, PK, NUM_KEYS)
  vidx, attn = _tc_score(x2d, Wq, g2d, b2d, Kt)
  out = _sc_bag(vidx, attn, values)
  return out.reshape(t, b, e)


# speedup vs baseline: 1.6146x; 1.3841x over previous
import functools

import jax
import jax.numpy as jnp
from jax import lax
from jax.experimental import pallas as pl
from jax.experimental.pallas import tpu as pltpu
from jax.experimental.pallas import tpu_sc as plsc

DIM = 1024
HEADS = 4
NUM_KEYS = 256
TOPK = 16
DIM_HEAD = 512
DIM_QUERY = DIM_HEAD * HEADS
T = 2048
PK = DIM_HEAD // 2
NJ = 2 * HEADS

TBLK = 256
NEG = -1e30


def _topk16(scores, payload=None):
  rows = scores.shape[0]
  iota_f = lax.broadcasted_iota(
      jnp.int32, (rows, NUM_KEYS), 1).astype(jnp.float32)
  cur = scores
  vs, is_, ps = [], [], []
  for _ in range(TOPK):
    m = jnp.max(cur, axis=1, keepdims=True)
    sel = cur == m
    pos = jnp.min(jnp.where(sel, iota_f, float(NUM_KEYS)), axis=1,
                  keepdims=True)
    hit = iota_f == pos
    vs.append(m)
    is_.append(pos)
    if payload is not None:
      ps.append(jnp.max(jnp.where(hit, payload, -1.0), axis=1, keepdims=True))
    cur = jnp.where(hit, NEG, cur)
  vals = jnp.concatenate(vs, axis=1)
  idx = jnp.concatenate(is_, axis=1)
  if payload is not None:
    return vals, idx, jnp.concatenate(ps, axis=1)
  return vals, idx


def _score_kernel(x_ref, wq_ref, g_ref, b_ref, kt_ref, vidx_ref, wexp_ref):
  q = jnp.dot(x_ref[...].astype(jnp.bfloat16),
              wq_ref[...].astype(jnp.bfloat16),
              preferred_element_type=jnp.float32)
  mu = jnp.mean(q, axis=1, keepdims=True)
  d = q - mu
  var = jnp.mean(d * d, axis=1, keepdims=True)
  q = d * lax.rsqrt(var + 1e-5) * g_ref[...] + b_ref[...]

  q16 = q.astype(jnp.bfloat16)
  s_list, i_list = [], []
  for j in range(NJ):
    dots = jnp.dot(q16[:, j * PK:(j + 1) * PK], kt_ref[j].astype(jnp.bfloat16),
                   preferred_element_type=jnp.float32)
    s, i_ = _topk16(dots)
    s_list.append(s)
    i_list.append(i_)

  li = lax.broadcasted_iota(jnp.int32, (TOPK, NUM_KEYS), 1)
  ri = lax.broadcasted_iota(jnp.int32, (TOPK, NUM_KEYS), 0)
  R = (li // TOPK == ri).astype(jnp.float32)
  C = (li % TOPK == ri).astype(jnp.float32)

  for h in range(HEADS):
    s0, s1 = s_list[h], s_list[HEADS + h]
    i0, i1 = i_list[h], i_list[HEADS + h]
    hi = lax.Precision.HIGHEST
    all_s = (jnp.dot(s0, R, preferred_element_type=jnp.float32, precision=hi)
             + jnp.dot(s1, C, preferred_element_type=jnp.float32, precision=hi))
    all_i = (jnp.dot(i0, R * float(NUM_KEYS), preferred_element_type=jnp.float32,
                     precision=hi)
             + jnp.dot(i1, C, preferred_element_type=jnp.float32, precision=hi))
    fs, _, fi = _topk16(all_s, payload=all_i)
    e = jnp.exp(fs - fs[:, :1])
    w = e / jnp.sum(e, axis=1, keepdims=True)
    vidx_ref[:, h * TOPK:(h + 1) * TOPK] = fi.astype(jnp.int32)
    wexp_ref[:, h * PK:(h + 1) * PK] = jnp.dot(
        w, R, preferred_element_type=jnp.float32, precision=lax.Precision.HIGHEST)


def _tc_score(x2d, Wq, g2d, b2d, Kt):
  nblk = T // TBLK
  return pl.pallas_call(
      _score_kernel,
      grid=(nblk,),
      in_specs=[
          pl.BlockSpec((TBLK, DIM), lambda i: (i, 0)),
          pl.BlockSpec((DIM, DIM_QUERY), lambda i: (0, 0)),
          pl.BlockSpec((1, DIM_QUERY), lambda i: (0, 0)),
          pl.BlockSpec((1, DIM_QUERY), lambda i: (0, 0)),
          pl.BlockSpec((NJ, PK, NUM_KEYS), lambda i: (0, 0, 0)),
      ],
      out_specs=[
          pl.BlockSpec((TBLK, HEADS * TOPK), lambda i: (i, 0)),
          pl.BlockSpec((TBLK, HEADS * TOPK * 16), lambda i: (i, 0)),
      ],
      out_shape=[
          jax.ShapeDtypeStruct((T, HEADS * TOPK), jnp.int32),
          jax.ShapeDtypeStruct((T, HEADS * TOPK * 16), jnp.float32),
      ],
  )(x2d, Wq, g2d, b2d, Kt)


K_PER_T = HEADS * TOPK


HALF_ROWS = K_PER_T // 2


def _bag_body(vidx_hbm, attn_hbm, values_hbm, out_hbm,
              idx2_v, w2_v, bufa_v, bufb_v, acc_v, sema, semb):
  nc = 2
  wid = lax.axis_index("s") * nc + lax.axis_index("c")
  t_per_w = T // 32
  t0 = wid * t_per_w

  def ga(parity):
    return pltpu.make_async_copy(
        values_hbm.at[idx2_v.at[parity, pl.ds(0, HALF_ROWS)]], bufa_v, sema)

  def gb(parity):
    return pltpu.make_async_copy(
        values_hbm.at[idx2_v.at[parity, pl.ds(HALF_ROWS, HALF_ROWS)]],
        bufb_v, semb)

  def accumulate(buf, p, lane0, init):
    out = []
    for c in range(4):
      def rstep(r, acc):
        wb = w2_v[p, pl.ds(lane0 + r * 16, 16)]
        return tuple(
            acc[dv] + wb * buf[r, pl.ds(c * 256 + dv * 16, 16)]
            for dv in range(16))
      if init is None:
        acc0 = tuple(jnp.zeros((16,), jnp.float32) for _ in range(16))
      else:
        acc0 = tuple(acc_v[pl.ds(c * 256 + dv * 16, 16)] for dv in range(16))
      acc = lax.fori_loop(0, HALF_ROWS, rstep, acc0)
      for dv in range(16):
        acc_v[pl.ds(c * 256 + dv * 16, 16)] = acc[dv]
    del out

  pltpu.sync_copy(vidx_hbm.at[t0], idx2_v.at[0])
  pltpu.sync_copy(attn_hbm.at[t0], w2_v.at[0])
  ga(0).start()

  def token(i, carry):
    t = t0 + i
    p = lax.rem(i, 2)
    pn = lax.rem(i + 1, 2)
    tn = t0 + lax.rem(i + 1, t_per_w)
    pltpu.sync_copy(vidx_hbm.at[tn], idx2_v.at[pn])
    pltpu.sync_copy(attn_hbm.at[tn], w2_v.at[pn])
    gb(p).start()
    ga(p).wait()
    accumulate(bufa_v, p, 0, init=None)
    ga(pn).start()
    gb(p).wait()
    accumulate(bufb_v, p, HALF_ROWS * 16, init=acc_v)
    pltpu.sync_copy(acc_v, out_hbm.at[t])
    return carry

  lax.fori_loop(0, t_per_w, token, 0)
  ga(0).wait()


def _sc_bag(vidx, attn, values):
  mesh = plsc.VectorSubcoreMesh(core_axis_name="c", subcore_axis_name="s")
  f = pl.kernel(
      _bag_body,
      out_type=jax.ShapeDtypeStruct((T, DIM), jnp.float32),
      mesh=mesh,
      scratch_types=[
          pltpu.VMEM((2, K_PER_T), jnp.int32),
          pltpu.VMEM((2, K_PER_T * 16), jnp.float32),
          pltpu.VMEM((HALF_ROWS, DIM), jnp.float32),
          pltpu.VMEM((HALF_ROWS, DIM), jnp.float32),
          pltpu.VMEM((DIM,), jnp.float32),
          pltpu.SemaphoreType.DMA,
          pltpu.SemaphoreType.DMA,
      ],
  )
  return f(vidx, attn, values)


def kernel(x, Wq, ln_g, ln_b, keys, values):
  t, b, e = x.shape
  x2d = x.reshape(t * b, e)
  g2d = ln_g.reshape(1, DIM_QUERY)
  b2d = ln_b.reshape(1, DIM_QUERY)
  Kt = jnp.transpose(keys, (2, 0, 3, 1)).reshape(NJ, PK, NUM_KEYS)
  vidx, attn = _tc_score(x2d, Wq, g2d, b2d, Kt)
  out = _sc_bag(vidx, attn, values)
  return out.reshape(t, b, e)


# speedup vs baseline: 1.9484x; 1.2068x over previous
import functools

import jax
import jax.numpy as jnp
from jax import lax
from jax.experimental import pallas as pl
from jax.experimental.pallas import tpu as pltpu
from jax.experimental.pallas import tpu_sc as plsc

DIM = 1024
HEADS = 4
NUM_KEYS = 256
TOPK = 16
DIM_HEAD = 512
DIM_QUERY = DIM_HEAD * HEADS
T = 2048
PK = DIM_HEAD // 2
NJ = 2 * HEADS

TBLK = 256
NEG = -1e30


def _topk16(scores, payload=None):
  rows = scores.shape[0]
  iota_f = lax.broadcasted_iota(
      jnp.int32, (rows, NUM_KEYS), 1).astype(jnp.float32)
  cur = scores
  vs, is_, ps = [], [], []
  for _ in range(TOPK):
    m = jnp.max(cur, axis=1, keepdims=True)
    sel = cur == m
    pos = jnp.min(jnp.where(sel, iota_f, float(NUM_KEYS)), axis=1,
                  keepdims=True)
    hit = iota_f == pos
    vs.append(m)
    is_.append(pos)
    if payload is not None:
      ps.append(jnp.max(jnp.where(hit, payload, -1.0), axis=1, keepdims=True))
    cur = jnp.where(hit, NEG, cur)
  vals = jnp.concatenate(vs, axis=1)
  idx = jnp.concatenate(is_, axis=1)
  if payload is not None:
    return vals, idx, jnp.concatenate(ps, axis=1)
  return vals, idx


def _score_kernel(x_ref, wq_ref, g_ref, b_ref, kt_ref, vidx_ref, wexp_ref):
  q = jnp.dot(x_ref[...].astype(jnp.bfloat16),
              wq_ref[...].astype(jnp.bfloat16),
              preferred_element_type=jnp.float32)
  mu = jnp.mean(q, axis=1, keepdims=True)
  d = q - mu
  var = jnp.mean(d * d, axis=1, keepdims=True)
  q = d * lax.rsqrt(var + 1e-5) * g_ref[...] + b_ref[...]

  q16 = q.astype(jnp.bfloat16)
  s_list, i_list = [], []
  for j in range(NJ):
    dots = jnp.dot(q16[:, j * PK:(j + 1) * PK], kt_ref[j].astype(jnp.bfloat16),
                   preferred_element_type=jnp.float32)
    s, i_ = _topk16(dots)
    s_list.append(s)
    i_list.append(i_)

  li = lax.broadcasted_iota(jnp.int32, (TOPK, NUM_KEYS), 1)
  ri = lax.broadcasted_iota(jnp.int32, (TOPK, NUM_KEYS), 0)
  R = (li // TOPK == ri).astype(jnp.float32)
  C = (li % TOPK == ri).astype(jnp.float32)

  for h in range(HEADS):
    s0, s1 = s_list[h], s_list[HEADS + h]
    i0, i1 = i_list[h], i_list[HEADS + h]
    hi = lax.Precision.HIGHEST
    all_s = (jnp.dot(s0, R, preferred_element_type=jnp.float32, precision=hi)
             + jnp.dot(s1, C, preferred_element_type=jnp.float32, precision=hi))
    all_i = (jnp.dot(i0, R * float(NUM_KEYS), preferred_element_type=jnp.float32,
                     precision=hi)
             + jnp.dot(i1, C, preferred_element_type=jnp.float32, precision=hi))
    fs, _, fi = _topk16(all_s, payload=all_i)
    e = jnp.exp(fs - fs[:, :1])
    w = e / jnp.sum(e, axis=1, keepdims=True)
    vidx_ref[:, h * TOPK:(h + 1) * TOPK] = fi.astype(jnp.int32)
    wexp_ref[:, h * PK:(h + 1) * PK] = jnp.dot(
        w, R, preferred_element_type=jnp.float32, precision=lax.Precision.HIGHEST)


def _tc_score(x2d, Wq, g2d, b2d, Kt):
  tcount = x2d.shape[0]
  nblk = tcount // TBLK
  return pl.pallas_call(
      _score_kernel,
      grid=(nblk,),
      in_specs=[
          pl.BlockSpec((TBLK, DIM), lambda i: (i, 0)),
          pl.BlockSpec((DIM, DIM_QUERY), lambda i: (0, 0)),
          pl.BlockSpec((1, DIM_QUERY), lambda i: (0, 0)),
          pl.BlockSpec((1, DIM_QUERY), lambda i: (0, 0)),
          pl.BlockSpec((NJ, PK, NUM_KEYS), lambda i: (0, 0, 0)),
      ],
      out_specs=[
          pl.BlockSpec((TBLK, HEADS * TOPK), lambda i: (i, 0)),
          pl.BlockSpec((TBLK, HEADS * TOPK * 16), lambda i: (i, 0)),
      ],
      out_shape=[
          jax.ShapeDtypeStruct((tcount, HEADS * TOPK), jnp.int32),
          jax.ShapeDtypeStruct((tcount, HEADS * TOPK * 16), jnp.float32),
      ],
  )(x2d, Wq, g2d, b2d, Kt)


K_PER_T = HEADS * TOPK


HALF_ROWS = K_PER_T // 2


def _bag_body(vidx_hbm, attn_hbm, values_hbm, out_hbm,
              idx2_v, w2_v, bufa_v, bufb_v, acc_v, sema, semb):
  nc = 2
  wid = lax.axis_index("s") * nc + lax.axis_index("c")
  t_per_w = vidx_hbm.shape[0] // 32
  t0 = wid * t_per_w

  def ga(parity):
    return pltpu.make_async_copy(
        values_hbm.at[idx2_v.at[parity, pl.ds(0, HALF_ROWS)]], bufa_v, sema)

  def gb(parity):
    return pltpu.make_async_copy(
        values_hbm.at[idx2_v.at[parity, pl.ds(HALF_ROWS, HALF_ROWS)]],
        bufb_v, semb)

  def accumulate(buf, p, lane0, init):
    out = []
    for c in range(4):
      def rstep(r, acc):
        wb = w2_v[p, pl.ds(lane0 + r * 16, 16)]
        return tuple(
            acc[dv] + wb * buf[r, pl.ds(c * 256 + dv * 16, 16)]
            for dv in range(16))
      if init is None:
        acc0 = tuple(jnp.zeros((16,), jnp.float32) for _ in range(16))
      else:
        acc0 = tuple(acc_v[pl.ds(c * 256 + dv * 16, 16)] for dv in range(16))
      acc = lax.fori_loop(0, HALF_ROWS, rstep, acc0)
      for dv in range(16):
        acc_v[pl.ds(c * 256 + dv * 16, 16)] = acc[dv]
    del out

  pltpu.sync_copy(vidx_hbm.at[t0], idx2_v.at[0])
  pltpu.sync_copy(attn_hbm.at[t0], w2_v.at[0])
  ga(0).start()

  def token(i, carry):
    t = t0 + i
    p = lax.rem(i, 2)
    pn = lax.rem(i + 1, 2)
    tn = t0 + lax.rem(i + 1, t_per_w)
    pltpu.sync_copy(vidx_hbm.at[tn], idx2_v.at[pn])
    pltpu.sync_copy(attn_hbm.at[tn], w2_v.at[pn])
    gb(p).start()
    ga(p).wait()
    accumulate(bufa_v, p, 0, init=None)
    ga(pn).start()
    gb(p).wait()
    accumulate(bufb_v, p, HALF_ROWS * 16, init=acc_v)
    pltpu.sync_copy(acc_v, out_hbm.at[t])
    return carry

  lax.fori_loop(0, t_per_w, token, 0)
  ga(0).wait()


def _sc_bag(vidx, attn, values):
  mesh = plsc.VectorSubcoreMesh(core_axis_name="c", subcore_axis_name="s")
  f = pl.kernel(
      _bag_body,
      out_type=jax.ShapeDtypeStruct((vidx.shape[0], DIM), jnp.float32),
      mesh=mesh,
      scratch_types=[
          pltpu.VMEM((2, K_PER_T), jnp.int32),
          pltpu.VMEM((2, K_PER_T * 16), jnp.float32),
          pltpu.VMEM((HALF_ROWS, DIM), jnp.float32),
          pltpu.VMEM((HALF_ROWS, DIM), jnp.float32),
          pltpu.VMEM((DIM,), jnp.float32),
          pltpu.SemaphoreType.DMA,
          pltpu.SemaphoreType.DMA,
      ],
  )
  return f(vidx, attn, values)


def kernel(x, Wq, ln_g, ln_b, keys, values):
  t, b, e = x.shape
  x2d = x.reshape(t * b, e)
  g2d = ln_g.reshape(1, DIM_QUERY)
  b2d = ln_b.reshape(1, DIM_QUERY)
  Kt = jnp.transpose(keys, (2, 0, 3, 1)).reshape(NJ, PK, NUM_KEYS)
  half = (t * b) // 2
  vidx0, attn0 = _tc_score(x2d[:half], Wq, g2d, b2d, Kt)
  out0 = _sc_bag(vidx0, attn0, values)
  vidx1, attn1 = _tc_score(x2d[half:], Wq, g2d, b2d, Kt)
  out1 = _sc_bag(vidx1, attn1, values)
  out = jnp.concatenate([out0, out1], axis=0)
  return out.reshape(t, b, e)


# speedup vs baseline: 2.0591x; 1.0568x over previous
import functools

import jax
import jax.numpy as jnp
from jax import lax
from jax.experimental import pallas as pl
from jax.experimental.pallas import tpu as pltpu
from jax.experimental.pallas import tpu_sc as plsc

DIM = 1024
HEADS = 4
NUM_KEYS = 256
TOPK = 16
DIM_HEAD = 512
DIM_QUERY = DIM_HEAD * HEADS
T = 2048
PK = DIM_HEAD // 2
NJ = 2 * HEADS

TBLK = 256
NEG = -1e30


def _topk16(scores, payload=None):
  rows = scores.shape[0]
  iota_f = lax.broadcasted_iota(
      jnp.int32, (rows, NUM_KEYS), 1).astype(jnp.float32)
  cur = scores
  vs, is_, ps = [], [], []
  for _ in range(TOPK):
    m = jnp.max(cur, axis=1, keepdims=True)
    sel = cur == m
    pos = jnp.min(jnp.where(sel, iota_f, float(NUM_KEYS)), axis=1,
                  keepdims=True)
    hit = iota_f == pos
    vs.append(m)
    is_.append(pos)
    if payload is not None:
      ps.append(jnp.max(jnp.where(hit, payload, -1.0), axis=1, keepdims=True))
    cur = jnp.where(hit, NEG, cur)
  vals = jnp.concatenate(vs, axis=1)
  idx = jnp.concatenate(is_, axis=1)
  if payload is not None:
    return vals, idx, jnp.concatenate(ps, axis=1)
  return vals, idx


def _score_kernel(x_ref, wq_ref, g_ref, b_ref, kt_ref, vidx_ref, wexp_ref):
  q = jnp.dot(x_ref[...].astype(jnp.bfloat16),
              wq_ref[...].astype(jnp.bfloat16),
              preferred_element_type=jnp.float32)
  mu = jnp.mean(q, axis=1, keepdims=True)
  d = q - mu
  var = jnp.mean(d * d, axis=1, keepdims=True)
  q = d * lax.rsqrt(var + 1e-5) * g_ref[...] + b_ref[...]

  q16 = q.astype(jnp.bfloat16)
  s_list, i_list = [], []
  for j in range(NJ):
    dots = jnp.dot(q16[:, j * PK:(j + 1) * PK], kt_ref[j].astype(jnp.bfloat16),
                   preferred_element_type=jnp.float32)
    s, i_ = _topk16(dots)
    s_list.append(s)
    i_list.append(i_)

  li = lax.broadcasted_iota(jnp.int32, (TOPK, NUM_KEYS), 1)
  ri = lax.broadcasted_iota(jnp.int32, (TOPK, NUM_KEYS), 0)
  R = (li // TOPK == ri).astype(jnp.float32)
  C = (li % TOPK == ri).astype(jnp.float32)

  for h in range(HEADS):
    s0, s1 = s_list[h], s_list[HEADS + h]
    i0, i1 = i_list[h], i_list[HEADS + h]
    hi = lax.Precision.HIGHEST
    all_s = (jnp.dot(s0, R, preferred_element_type=jnp.float32, precision=hi)
             + jnp.dot(s1, C, preferred_element_type=jnp.float32, precision=hi))
    all_i = (jnp.dot(i0, R * float(NUM_KEYS), preferred_element_type=jnp.float32,
                     precision=hi)
             + jnp.dot(i1, C, preferred_element_type=jnp.float32, precision=hi))
    fs, _, fi = _topk16(all_s, payload=all_i)
    e = jnp.exp(fs - fs[:, :1])
    w = e / jnp.sum(e, axis=1, keepdims=True)
    vidx_ref[:, h * TOPK:(h + 1) * TOPK] = fi.astype(jnp.int32)
    wexp_ref[:, h * PK:(h + 1) * PK] = jnp.dot(
        w, R, preferred_element_type=jnp.float32, precision=lax.Precision.HIGHEST)


def _tc_score(x2d, Wq, g2d, b2d, Kt):
  tcount = x2d.shape[0]
  nblk = tcount // TBLK
  return pl.pallas_call(
      _score_kernel,
      grid=(nblk,),
      in_specs=[
          pl.BlockSpec((TBLK, DIM), lambda i: (i, 0)),
          pl.BlockSpec((DIM, DIM_QUERY), lambda i: (0, 0)),
          pl.BlockSpec((1, DIM_QUERY), lambda i: (0, 0)),
          pl.BlockSpec((1, DIM_QUERY), lambda i: (0, 0)),
          pl.BlockSpec((NJ, PK, NUM_KEYS), lambda i: (0, 0, 0)),
      ],
      out_specs=[
          pl.BlockSpec((TBLK, HEADS * TOPK), lambda i: (i, 0)),
          pl.BlockSpec((TBLK, HEADS * TOPK * 16), lambda i: (i, 0)),
      ],
      out_shape=[
          jax.ShapeDtypeStruct((tcount, HEADS * TOPK), jnp.int32),
          jax.ShapeDtypeStruct((tcount, HEADS * TOPK * 16), jnp.float32),
      ],
  )(x2d, Wq, g2d, b2d, Kt)


K_PER_T = HEADS * TOPK


HALF_ROWS = K_PER_T // 2


def _bag_body(vidx_hbm, attn_hbm, values_hbm, out_hbm,
              idx2_v, w2_v, bufa_v, bufb_v, acc_v, sema, semb):
  nc = 2
  wid = lax.axis_index("s") * nc + lax.axis_index("c")
  t_per_w = vidx_hbm.shape[0] // 32
  t0 = wid * t_per_w

  def ga(parity):
    return pltpu.make_async_copy(
        values_hbm.at[idx2_v.at[parity, pl.ds(0, HALF_ROWS)]], bufa_v, sema)

  def gb(parity):
    return pltpu.make_async_copy(
        values_hbm.at[idx2_v.at[parity, pl.ds(HALF_ROWS, HALF_ROWS)]],
        bufb_v, semb)

  def accumulate(buf, p, lane0, init):
    out = []
    for c in range(4):
      def rstep(r, acc):
        wb = w2_v[p, pl.ds(lane0 + r * 16, 16)]
        return tuple(
            acc[dv] + wb * buf[r, pl.ds(c * 256 + dv * 16, 16)]
            for dv in range(16))
      if init is None:
        acc0 = tuple(jnp.zeros((16,), jnp.float32) for _ in range(16))
      else:
        acc0 = tuple(acc_v[pl.ds(c * 256 + dv * 16, 16)] for dv in range(16))
      acc = lax.fori_loop(0, HALF_ROWS, rstep, acc0)
      for dv in range(16):
        acc_v[pl.ds(c * 256 + dv * 16, 16)] = acc[dv]
    del out

  pltpu.sync_copy(vidx_hbm.at[t0], idx2_v.at[0])
  pltpu.sync_copy(attn_hbm.at[t0], w2_v.at[0])
  ga(0).start()

  def token(i, carry):
    t = t0 + i
    p = lax.rem(i, 2)
    pn = lax.rem(i + 1, 2)
    tn = t0 + lax.rem(i + 1, t_per_w)
    pltpu.sync_copy(vidx_hbm.at[tn], idx2_v.at[pn])
    pltpu.sync_copy(attn_hbm.at[tn], w2_v.at[pn])
    gb(p).start()
    ga(p).wait()
    accumulate(bufa_v, p, 0, init=None)
    ga(pn).start()
    gb(p).wait()
    accumulate(bufb_v, p, HALF_ROWS * 16, init=acc_v)
    pltpu.sync_copy(acc_v, out_hbm.at[t])
    return carry

  lax.fori_loop(0, t_per_w, token, 0)
  ga(0).wait()


def _sc_bag(vidx, attn, values):
  mesh = plsc.VectorSubcoreMesh(core_axis_name="c", subcore_axis_name="s")
  f = pl.kernel(
      _bag_body,
      out_type=jax.ShapeDtypeStruct((vidx.shape[0], DIM), jnp.float32),
      mesh=mesh,
      scratch_types=[
          pltpu.VMEM((2, K_PER_T), jnp.int32),
          pltpu.VMEM((2, K_PER_T * 16), jnp.float32),
          pltpu.VMEM((HALF_ROWS, DIM), jnp.float32),
          pltpu.VMEM((HALF_ROWS, DIM), jnp.float32),
          pltpu.VMEM((DIM,), jnp.float32),
          pltpu.SemaphoreType.DMA,
          pltpu.SemaphoreType.DMA,
      ],
  )
  return f(vidx, attn, values)


def kernel(x, Wq, ln_g, ln_b, keys, values):
  t, b, e = x.shape
  x2d = x.reshape(t * b, e)
  g2d = ln_g.reshape(1, DIM_QUERY)
  b2d = ln_b.reshape(1, DIM_QUERY)
  Kt = jnp.transpose(keys, (2, 0, 3, 1)).reshape(NJ, PK, NUM_KEYS)
  nsplit = 4
  piece = (t * b) // nsplit
  outs = []
  for i in range(nsplit):
    vidx_i, attn_i = _tc_score(x2d[i * piece:(i + 1) * piece], Wq, g2d, b2d, Kt)
    outs.append(_sc_bag(vidx_i, attn_i, values))
  out = jnp.concatenate(outs, axis=0)
  return out.reshape(t, b, e)


# speedup vs baseline: 2.0680x; 1.0043x over previous
import functools

import jax
import jax.numpy as jnp
from jax import lax
from jax.experimental import pallas as pl
from jax.experimental.pallas import tpu as pltpu
from jax.experimental.pallas import tpu_sc as plsc

DIM = 1024
HEADS = 4
NUM_KEYS = 256
TOPK = 16
DIM_HEAD = 512
DIM_QUERY = DIM_HEAD * HEADS
T = 2048
PK = DIM_HEAD // 2
NJ = 2 * HEADS

TBLK = 256
NEG = -1e30


def _topk16(scores, payload=None):
  rows = scores.shape[0]
  iota_f = lax.broadcasted_iota(
      jnp.int32, (rows, NUM_KEYS), 1).astype(jnp.float32)
  cur = scores
  vs, is_, ps = [], [], []
  for _ in range(TOPK):
    m = jnp.max(cur, axis=1, keepdims=True)
    sel = cur == m
    pos = jnp.min(jnp.where(sel, iota_f, float(NUM_KEYS)), axis=1,
                  keepdims=True)
    hit = iota_f == pos
    vs.append(m)
    is_.append(pos)
    if payload is not None:
      ps.append(jnp.max(jnp.where(hit, payload, -1.0), axis=1, keepdims=True))
    cur = jnp.where(hit, NEG, cur)
  vals = jnp.concatenate(vs, axis=1)
  idx = jnp.concatenate(is_, axis=1)
  if payload is not None:
    return vals, idx, jnp.concatenate(ps, axis=1)
  return vals, idx


def _score_kernel(x_ref, wq_ref, g_ref, b_ref, kt_ref, vidx_ref, wexp_ref):
  q = jnp.dot(x_ref[...].astype(jnp.bfloat16),
              wq_ref[...].astype(jnp.bfloat16),
              preferred_element_type=jnp.float32)
  mu = jnp.mean(q, axis=1, keepdims=True)
  d = q - mu
  var = jnp.mean(d * d, axis=1, keepdims=True)
  q = d * lax.rsqrt(var + 1e-5) * g_ref[...] + b_ref[...]

  q16 = q.astype(jnp.bfloat16)
  s_list, i_list = [], []
  for j in range(NJ):
    dots = jnp.dot(q16[:, j * PK:(j + 1) * PK], kt_ref[j].astype(jnp.bfloat16),
                   preferred_element_type=jnp.float32)
    s, i_ = _topk16(dots)
    s_list.append(s)
    i_list.append(i_)

  li = lax.broadcasted_iota(jnp.int32, (TOPK, NUM_KEYS), 1)
  ri = lax.broadcasted_iota(jnp.int32, (TOPK, NUM_KEYS), 0)
  R = (li // TOPK == ri).astype(jnp.float32)
  C = (li % TOPK == ri).astype(jnp.float32)

  for h in range(HEADS):
    s0, s1 = s_list[h], s_list[HEADS + h]
    i0, i1 = i_list[h], i_list[HEADS + h]
    hi = lax.Precision.HIGHEST
    all_s = (jnp.dot(s0, R, preferred_element_type=jnp.float32, precision=hi)
             + jnp.dot(s1, C, preferred_element_type=jnp.float32, precision=hi))
    all_i = (jnp.dot(i0, R * float(NUM_KEYS), preferred_element_type=jnp.float32,
                     precision=hi)
             + jnp.dot(i1, C, preferred_element_type=jnp.float32, precision=hi))
    fs, _, fi = _topk16(all_s, payload=all_i)
    e = jnp.exp(fs - fs[:, :1])
    w = e / jnp.sum(e, axis=1, keepdims=True)
    vidx_ref[:, h * TOPK:(h + 1) * TOPK] = fi.astype(jnp.int32)
    wexp_ref[:, h * PK:(h + 1) * PK] = jnp.dot(
        w, R, preferred_element_type=jnp.float32, precision=lax.Precision.HIGHEST)


def _tc_score(x2d, Wq, g2d, b2d, Kt):
  tcount = x2d.shape[0]
  nblk = tcount // TBLK
  return pl.pallas_call(
      _score_kernel,
      grid=(nblk,),
      in_specs=[
          pl.BlockSpec((TBLK, DIM), lambda i: (i, 0)),
          pl.BlockSpec((DIM, DIM_QUERY), lambda i: (0, 0)),
          pl.BlockSpec((1, DIM_QUERY), lambda i: (0, 0)),
          pl.BlockSpec((1, DIM_QUERY), lambda i: (0, 0)),
          pl.BlockSpec((NJ, PK, NUM_KEYS), lambda i: (0, 0, 0)),
      ],
      out_specs=[
          pl.BlockSpec((TBLK, HEADS * TOPK), lambda i: (i, 0)),
          pl.BlockSpec((TBLK, HEADS * TOPK * 16), lambda i: (i, 0)),
      ],
      out_shape=[
          jax.ShapeDtypeStruct((tcount, HEADS * TOPK), jnp.int32),
          jax.ShapeDtypeStruct((tcount, HEADS * TOPK * 16), jnp.float32),
      ],
  )(x2d, Wq, g2d, b2d, Kt)


K_PER_T = HEADS * TOPK


HALF_ROWS = K_PER_T // 2


def _bag_body(vidx_hbm, attn_hbm, values_hbm, out_hbm,
              idx2_v, w2_v, bufa_v, bufb_v, acc_v, sema, semb):
  nc = 2
  wid = lax.axis_index("s") * nc + lax.axis_index("c")
  t_per_w = vidx_hbm.shape[0] // 32
  t0 = wid * t_per_w

  def ga(parity):
    return pltpu.make_async_copy(
        values_hbm.at[idx2_v.at[parity, pl.ds(0, HALF_ROWS)]], bufa_v, sema)

  def gb(parity):
    return pltpu.make_async_copy(
        values_hbm.at[idx2_v.at[parity, pl.ds(HALF_ROWS, HALF_ROWS)]],
        bufb_v, semb)

  def accumulate(buf, p, lane0, init):
    out = []
    for c in range(4):
      def rstep(r, acc):
        wb = w2_v[p, pl.ds(lane0 + r * 16, 16)]
        return tuple(
            acc[dv] + wb * buf[r, pl.ds(c * 256 + dv * 16, 16)]
            for dv in range(16))
      if init is None:
        acc0 = tuple(jnp.zeros((16,), jnp.float32) for _ in range(16))
      else:
        acc0 = tuple(acc_v[pl.ds(c * 256 + dv * 16, 16)] for dv in range(16))
      acc = lax.fori_loop(0, HALF_ROWS, rstep, acc0)
      for dv in range(16):
        acc_v[pl.ds(c * 256 + dv * 16, 16)] = acc[dv]
    del out

  pltpu.sync_copy(vidx_hbm.at[t0], idx2_v.at[0])
  pltpu.sync_copy(attn_hbm.at[t0], w2_v.at[0])
  ga(0).start()

  def token(i, carry):
    t = t0 + i
    p = lax.rem(i, 2)
    pn = lax.rem(i + 1, 2)
    tn = t0 + lax.rem(i + 1, t_per_w)
    pltpu.sync_copy(vidx_hbm.at[tn], idx2_v.at[pn])
    pltpu.sync_copy(attn_hbm.at[tn], w2_v.at[pn])
    gb(p).start()
    ga(p).wait()
    accumulate(bufa_v, p, 0, init=None)
    ga(pn).start()
    gb(p).wait()
    accumulate(bufb_v, p, HALF_ROWS * 16, init=acc_v)
    pltpu.sync_copy(acc_v, out_hbm.at[t])
    return carry

  lax.fori_loop(0, t_per_w, token, 0)
  ga(0).wait()


def _sc_bag(vidx, attn, values):
  mesh = plsc.VectorSubcoreMesh(core_axis_name="c", subcore_axis_name="s")
  f = pl.kernel(
      _bag_body,
      out_type=jax.ShapeDtypeStruct((vidx.shape[0], DIM), jnp.float32),
      mesh=mesh,
      scratch_types=[
          pltpu.VMEM((2, K_PER_T), jnp.int32),
          pltpu.VMEM((2, K_PER_T * 16), jnp.float32),
          pltpu.VMEM((HALF_ROWS, DIM), jnp.float32),
          pltpu.VMEM((HALF_ROWS, DIM), jnp.float32),
          pltpu.VMEM((DIM,), jnp.float32),
          pltpu.SemaphoreType.DMA,
          pltpu.SemaphoreType.DMA,
      ],
  )
  return f(vidx, attn, values)


def kernel(x, Wq, ln_g, ln_b, keys, values):
  t, b, e = x.shape
  x2d = x.reshape(t * b, e)
  g2d = ln_g.reshape(1, DIM_QUERY)
  b2d = ln_b.reshape(1, DIM_QUERY)
  Kt = jnp.transpose(keys, (2, 0, 3, 1)).reshape(NJ, PK, NUM_KEYS)
  nsplit = 8
  piece = (t * b) // nsplit
  outs = []
  for i in range(nsplit):
    vidx_i, attn_i = _tc_score(x2d[i * piece:(i + 1) * piece], Wq, g2d, b2d, Kt)
    outs.append(_sc_bag(vidx_i, attn_i, values))
  out = jnp.concatenate(outs, axis=0)
  return out.reshape(t, b, e)
